# baseline (speedup 1.0000x reference)
import functools

import jax
import jax.numpy as jnp
from jax import lax
from jax.experimental import pallas as pl
from jax.experimental.pallas import tpu as pltpu
from jax.experimental.pallas import tpu_sc as plsc

BATCH = 16384
EMBED = 32
NC, NS = 2, 16
NW = NC * NS
B_PER_W = BATCH // NW
CHUNK = 128
NCHUNK = B_PER_W // CHUNK


def _sc_gather_body(uidx_hbm, iidx_hbm, utab_hbm, itab_hbm,
                    uout_hbm, iout_hbm,
                    uidx_v, iidx_v, urows_v, irows_v, sem):
    wid = lax.axis_index("s") * NC + lax.axis_index("c")
    base = wid * B_PER_W
    pltpu.sync_copy(uidx_hbm.at[pl.ds(base, B_PER_W)], uidx_v)
    pltpu.sync_copy(iidx_hbm.at[pl.ds(base, B_PER_W)], iidx_v)
    copies = []
    for j in range(NCHUNK):
        sl = pl.ds(j * CHUNK, CHUNK)
        copies.append(pltpu.async_copy(utab_hbm.at[uidx_v.at[sl]], urows_v.at[sl], sem))
        copies.append(pltpu.async_copy(itab_hbm.at[iidx_v.at[sl]], irows_v.at[sl], sem))
    for c in copies:
        c.wait()
    pltpu.sync_copy(urows_v, uout_hbm.at[pl.ds(base, B_PER_W)])
    pltpu.sync_copy(irows_v, iout_hbm.at[pl.ds(base, B_PER_W)])


@jax.jit
def _sc_gather(user, item, user_table, item_table):
    mesh = plsc.VectorSubcoreMesh(core_axis_name="c", subcore_axis_name="s")
    k = functools.partial(
        pl.kernel,
        mesh=mesh,
        compiler_params=pltpu.CompilerParams(use_tc_tiling_on_sc=False),
        out_type=[jax.ShapeDtypeStruct((BATCH, EMBED), jnp.float32),
                  jax.ShapeDtypeStruct((BATCH, EMBED), jnp.float32)],
        scratch_types=[
            pltpu.VMEM((B_PER_W,), jnp.int32),
            pltpu.VMEM((B_PER_W,), jnp.int32),
            pltpu.VMEM((B_PER_W, EMBED), jnp.float32),
            pltpu.VMEM((B_PER_W, EMBED), jnp.float32),
            pltpu.SemaphoreType.DMA,
        ],
    )(_sc_gather_body)
    return k(user, item, user_table, item_table)


def _mlp_body(u_ref, i_ref, w1_ref, b1_ref, w2_ref, b2_ref, o_ref):
    x = jnp.dot(u_ref[...], w1_ref[0:EMBED, :], preferred_element_type=jnp.float32)
    x = x + jnp.dot(i_ref[...], w1_ref[EMBED:2 * EMBED, :],
                    preferred_element_type=jnp.float32)
    x = jnp.maximum(x + b1_ref[...], 0.0)
    o_ref[...] = jnp.dot(x, w2_ref[...], preferred_element_type=jnp.float32) + b2_ref[...]


@jax.jit
def _tc_mlp(u_emb, i_emb, W1, b1, W2, b2):
    R = 2048
    grid = (BATCH // R,)
    return pl.pallas_call(
        _mlp_body,
        grid=grid,
        in_specs=[
            pl.BlockSpec((R, EMBED), lambda r: (r, 0)),
            pl.BlockSpec((R, EMBED), lambda r: (r, 0)),
            pl.BlockSpec((2 * EMBED, 128), lambda r: (0, 0)),
            pl.BlockSpec((1, 128), lambda r: (0, 0)),
            pl.BlockSpec((128, 5), lambda r: (0, 0)),
            pl.BlockSpec((1, 5), lambda r: (0, 0)),
        ],
        out_specs=pl.BlockSpec((R, 5), lambda r: (r, 0)),
        out_shape=jax.ShapeDtypeStruct((BATCH, 5), jnp.float32),
    )(u_emb, i_emb, W1, b1.reshape(1, 128), W2, b2.reshape(1, 5))


def kernel(user, item, user_table, item_table, W1, b1, W2, b2):
    u_emb, i_emb = _sc_gather(user, item, user_table, item_table)
    return _tc_mlp(u_emb, i_emb, W1, b1, W2, b2)
